# baseline (speedup 1.0000x reference)
import functools

import jax
import jax.numpy as jnp
from jax import lax
from jax.experimental import pallas as pl
from jax.experimental.pallas import tpu as pltpu
from jax.experimental.pallas import tpu_sc as plsc

B_SIZE = 16384
V_SIZE = 1000000
D = 32

NC = 2
NS = 16
NW = NC * NS
BPW = B_SIZE // NW
CH = 128
NCH = BPW // CH

_sc_mesh = plsc.VectorSubcoreMesh(core_axis_name="c", subcore_axis_name="s")


@functools.partial(
    pl.kernel,
    out_type=[
        jax.ShapeDtypeStruct((B_SIZE, D), jnp.float32),
        jax.ShapeDtypeStruct((B_SIZE, D), jnp.float32),
        jax.ShapeDtypeStruct((B_SIZE, 1), jnp.float32),
        jax.ShapeDtypeStruct((B_SIZE, 1), jnp.float32),
    ],
    mesh=_sc_mesh,
    compiler_params=pltpu.CompilerParams(use_tc_tiling_on_sc=False),
    scratch_types=[
        pltpu.VMEM((NCH, CH), jnp.int32),
        pltpu.VMEM((NCH, CH), jnp.int32),
        pltpu.VMEM((BPW, D), jnp.float32),
        pltpu.VMEM((BPW, D), jnp.float32),
        pltpu.VMEM((BPW, 1), jnp.float32),
        pltpu.VMEM((BPW, 1), jnp.float32),
        pltpu.SemaphoreType.DMA,
        pltpu.SemaphoreType.DMA,
    ],
)
def _sc_gather(uid_hbm, iid_hbm, u_tab, q_tab, a_tab, b_tab,
               u_out, q_out, a_out, b_out,
               uidx, iidx, urows, qrows, arows, brows, sem_in, sem_out):
    wid = lax.axis_index("s") * NC + lax.axis_index("c")
    base = wid * BPW

    pltpu.sync_copy(uid_hbm.at[wid], uidx)
    pltpu.sync_copy(iid_hbm.at[wid], iidx)

    handles = []
    for j in range(NCH):
        rows = pl.ds(j * CH, CH)
        handles.append(pltpu.async_copy(u_tab.at[uidx.at[j]], urows.at[rows], sem_in))
        handles.append(pltpu.async_copy(q_tab.at[iidx.at[j]], qrows.at[rows], sem_in))
        handles.append(pltpu.async_copy(a_tab.at[uidx.at[j]], arows.at[rows], sem_in))
        handles.append(pltpu.async_copy(b_tab.at[iidx.at[j]], brows.at[rows], sem_in))
    for h in handles:
        h.wait()

    out = pl.ds(base, BPW)
    wb = [
        pltpu.async_copy(urows, u_out.at[out], sem_out),
        pltpu.async_copy(qrows, q_out.at[out], sem_out),
        pltpu.async_copy(arows, a_out.at[out], sem_out),
        pltpu.async_copy(brows, b_out.at[out], sem_out),
    ]
    for h in wb:
        h.wait()


BLK = 2048


def _tc_body(u_ref, q_ref, a_ref, b_ref, w1t_ref, b1_ref, w2t_ref, b2_ref,
             pred_ref, score_ref):
    u = u_ref[...]
    q = q_ref[...]
    uq = u * q
    pred_ref[...] = (jnp.sum(uq, axis=1, keepdims=True)
                     + a_ref[...] + b_ref[...])
    h = jnp.concatenate([u, q, uq], axis=1)
    h = jnp.dot(h, w1t_ref[...], preferred_element_type=jnp.float32)
    h = jnp.maximum(h + b1_ref[...], 0.0)
    s = jnp.dot(h, w2t_ref[...], preferred_element_type=jnp.float32)
    score_ref[...] = s + b2_ref[...]


_tc_mlp = pl.pallas_call(
    _tc_body,
    grid=(B_SIZE // BLK,),
    in_specs=[
        pl.BlockSpec((BLK, D), lambda i: (i, 0)),
        pl.BlockSpec((BLK, D), lambda i: (i, 0)),
        pl.BlockSpec((BLK, 1), lambda i: (i, 0)),
        pl.BlockSpec((BLK, 1), lambda i: (i, 0)),
        pl.BlockSpec((3 * D, 64), lambda i: (0, 0)),
        pl.BlockSpec((1, 64), lambda i: (0, 0)),
        pl.BlockSpec((64, 1), lambda i: (0, 0)),
        pl.BlockSpec((1, 1), lambda i: (0, 0)),
    ],
    out_specs=[
        pl.BlockSpec((BLK, 1), lambda i: (i, 0)),
        pl.BlockSpec((BLK, 1), lambda i: (i, 0)),
    ],
    out_shape=[
        jax.ShapeDtypeStruct((B_SIZE, 1), jnp.float32),
        jax.ShapeDtypeStruct((B_SIZE, 1), jnp.float32),
    ],
)


@jax.jit
def kernel(user_ids, item_ids, U_mf, Q_mf, A_mf, B_mf, W1, b1, W2, b2):
    uid = user_ids.astype(jnp.int32).reshape(NW, NCH, CH)
    iid = item_ids.astype(jnp.int32).reshape(NW, NCH, CH)
    u, q, a2, bb2 = _sc_gather(uid, iid, U_mf, Q_mf, A_mf, B_mf)
    pred2, score2 = _tc_mlp(u, q, a2, bb2,
                            W1.T, b1.reshape(1, 64), W2.T, b2.reshape(1, 1))
    return pred2[:, 0], score2[:, 0]


# speedup vs baseline: 2.6852x; 2.6852x over previous
import functools

import jax
import jax.numpy as jnp
from jax import lax
from jax.experimental import pallas as pl
from jax.experimental.pallas import tpu as pltpu
from jax.experimental.pallas import tpu_sc as plsc

B_SIZE = 16384
V_SIZE = 1000000
D = 32
WIDE = 128
PACK = WIDE // D

NC = 2
NS = 16
NW = NC * NS
BPW = B_SIZE // NW
CH = 128
NCH = BPW // CH

_sc_mesh = plsc.VectorSubcoreMesh(core_axis_name="c", subcore_axis_name="s")


@functools.partial(
    pl.kernel,
    out_type=[
        jax.ShapeDtypeStruct((B_SIZE, WIDE), jnp.float32),
        jax.ShapeDtypeStruct((B_SIZE, WIDE), jnp.float32),
        jax.ShapeDtypeStruct((B_SIZE,), jnp.float32),
        jax.ShapeDtypeStruct((B_SIZE,), jnp.float32),
    ],
    mesh=_sc_mesh,
    scratch_types=[
        pltpu.VMEM((NCH, CH), jnp.int32),
        pltpu.VMEM((NCH, CH), jnp.int32),
        pltpu.VMEM((NCH, CH), jnp.int32),
        pltpu.VMEM((NCH, CH), jnp.int32),
        pltpu.VMEM((BPW // 2, WIDE), jnp.float32),
        pltpu.VMEM((BPW // 2, WIDE), jnp.float32),
        pltpu.VMEM((BPW,), jnp.float32),
        pltpu.VMEM((BPW,), jnp.float32),
        pltpu.SemaphoreType.DMA,
        pltpu.SemaphoreType.DMA,
    ],
)
def _sc_gather(uid_hbm, iid_hbm, uw_tab, qw_tab, a_tab, b_tab,
               u_out, q_out, a_out, b_out,
               uidx, iidx, uwidx, iwidx, urows, qrows, avals, bvals,
               sem_in, sem_out):
    wid = lax.axis_index("s") * NC + lax.axis_index("c")
    base = wid * BPW

    pltpu.sync_copy(uid_hbm.at[wid], uidx)
    pltpu.sync_copy(iid_hbm.at[wid], iidx)

    for j in range(NCH):
        for k in range(CH // 16):
            s = pl.ds(k * 16, 16)
            uwidx[j, s] = lax.shift_right_logical(uidx[j, s], 2)
            iwidx[j, s] = lax.shift_right_logical(iidx[j, s], 2)

    bias_handles = []
    for j in range(NCH):
        rows = pl.ds(j * CH, CH)
        bias_handles.append(pltpu.async_copy(a_tab.at[uidx.at[j]], avals.at[rows], sem_in))
        bias_handles.append(pltpu.async_copy(b_tab.at[iidx.at[j]], bvals.at[rows], sem_in))

    for half in range(2):
        handles = []
        for jj in range(NCH // 2):
            j = half * (NCH // 2) + jj
            rows = pl.ds(jj * CH, CH)
            handles.append(pltpu.async_copy(uw_tab.at[uwidx.at[j]], urows.at[rows], sem_in))
            handles.append(pltpu.async_copy(qw_tab.at[iwidx.at[j]], qrows.at[rows], sem_in))
        for h in handles:
            h.wait()
        out = pl.ds(base + half * (BPW // 2), BPW // 2)
        pltpu.async_copy(urows, u_out.at[out], sem_out).wait()
        pltpu.async_copy(qrows, q_out.at[out], sem_out).wait()

    for h in bias_handles:
        h.wait()
    out = pl.ds(base, BPW)
    pltpu.async_copy(avals, a_out.at[out], sem_out).wait()
    pltpu.async_copy(bvals, b_out.at[out], sem_out).wait()


BLK = 2048


def _tc_body(uw_ref, qw_ref, a_ref, b_ref, uid_ref, iid_ref,
             w1t_ref, b1_ref, w2t_ref, b2_ref, pred_ref, score_ref):
    uw = uw_ref[...]
    qw = qw_ref[...]
    usel = uid_ref[...] & (PACK - 1)
    isel = iid_ref[...] & (PACK - 1)
    u = jnp.zeros((BLK, D), jnp.float32)
    q = jnp.zeros((BLK, D), jnp.float32)
    for k in range(PACK):
        u = u + jnp.where(usel == k, uw[:, k * D:(k + 1) * D], 0.0)
        q = q + jnp.where(isel == k, qw[:, k * D:(k + 1) * D], 0.0)
    uq = u * q
    pred_ref[...] = (jnp.sum(uq, axis=1, keepdims=True)
                     + a_ref[...] + b_ref[...])
    h = jnp.concatenate([u, q, uq], axis=1)
    h = jnp.dot(h, w1t_ref[...], preferred_element_type=jnp.float32)
    h = jnp.maximum(h + b1_ref[...], 0.0)
    s = jnp.dot(h, w2t_ref[...], preferred_element_type=jnp.float32)
    score_ref[...] = s + b2_ref[...]


_tc_mlp = pl.pallas_call(
    _tc_body,
    grid=(B_SIZE // BLK,),
    in_specs=[
        pl.BlockSpec((BLK, WIDE), lambda i: (i, 0)),
        pl.BlockSpec((BLK, WIDE), lambda i: (i, 0)),
        pl.BlockSpec((BLK, 1), lambda i: (i, 0)),
        pl.BlockSpec((BLK, 1), lambda i: (i, 0)),
        pl.BlockSpec((BLK, 1), lambda i: (i, 0)),
        pl.BlockSpec((BLK, 1), lambda i: (i, 0)),
        pl.BlockSpec((3 * D, 64), lambda i: (0, 0)),
        pl.BlockSpec((1, 64), lambda i: (0, 0)),
        pl.BlockSpec((64, 1), lambda i: (0, 0)),
        pl.BlockSpec((1, 1), lambda i: (0, 0)),
    ],
    out_specs=[
        pl.BlockSpec((BLK, 1), lambda i: (i, 0)),
        pl.BlockSpec((BLK, 1), lambda i: (i, 0)),
    ],
    out_shape=[
        jax.ShapeDtypeStruct((B_SIZE, 1), jnp.float32),
        jax.ShapeDtypeStruct((B_SIZE, 1), jnp.float32),
    ],
)


@jax.jit
def kernel(user_ids, item_ids, U_mf, Q_mf, A_mf, B_mf, W1, b1, W2, b2):
    uid = user_ids.astype(jnp.int32)
    iid = item_ids.astype(jnp.int32)
    uid3 = uid.reshape(NW, NCH, CH)
    iid3 = iid.reshape(NW, NCH, CH)
    uw_tab = U_mf.reshape(V_SIZE // PACK, WIDE)
    qw_tab = Q_mf.reshape(V_SIZE // PACK, WIDE)
    a_tab = A_mf.reshape(V_SIZE)
    b_tab = B_mf.reshape(V_SIZE)
    u, q, a, b = _sc_gather(uid3, iid3, uw_tab, qw_tab, a_tab, b_tab)
    pred2, score2 = _tc_mlp(u, q, a.reshape(B_SIZE, 1), b.reshape(B_SIZE, 1),
                            uid.reshape(B_SIZE, 1), iid.reshape(B_SIZE, 1),
                            W1.T, b1.reshape(1, 64), W2.T, b2.reshape(1, 1))
    return pred2[:, 0], score2[:, 0]
